# Initial kernel scaffold; baseline (speedup 1.0000x reference)
import jax
import jax.numpy as jnp
from jax.experimental import pallas as pl


def kernel(inputs, table):
    raise NotImplementedError("write your pallas kernel here")


# speedup vs baseline: 1.5603x; 1.5603x over previous
import functools

import jax
import jax.numpy as jnp
import numpy as np
from jax import lax
from jax.experimental import pallas as pl
from jax.experimental.pallas import tpu as pltpu
from jax.experimental.pallas import tpu_sc as plsc

_D = 64
_B = 16384
_H = 50
_TOTAL = _B * _H
_NC, _NS = 2, 16
_NW = _NC * _NS
_PER_W = _TOTAL // _NW
_CHUNK = 128
_NCHUNK = _PER_W // _CHUNK

_K = 2.0
_IR = np.float32(2.0 * _K / (1.0 + np.sqrt(1.0 + 4.0 * _K * _K)))
_EPS = np.float32(1e-05)


def _rsqrt(x):
    i = plsc.bitcast(x, jnp.int32)
    i = jnp.int32(0x5F3759DF) - (i >> 1)
    y = plsc.bitcast(i, jnp.float32)
    for _ in range(3):
        y = y * (jnp.float32(1.5) - jnp.float32(0.5) * x * y * y)
    return y


def _soft_clip_scale(sumsq):
    y = _rsqrt(sumsq)
    norm = sumsq * y
    one = jnp.full((16,), 1.0, jnp.float32)
    return jnp.where(norm <= _IR, _IR * y,
                     jnp.where(norm >= jnp.float32(1.0), (jnp.float32(1.0) - _EPS) * y,
                               one))


def _embed_body(idx_hbm, table_hbm, out_hbm, idx_v, in_v, out_v, sem_in):
    wid = lax.axis_index("s") * _NC + lax.axis_index("c")
    base = wid * _PER_W
    pltpu.sync_copy(idx_hbm.at[pl.ds(base, _PER_W)], idx_v)

    def chunk_body(c, carry):
        pltpu.async_copy(table_hbm.at[idx_v.at[pl.ds(c * _CHUNK, _CHUNK)]],
                         in_v, sem_in).wait()

        def quad_body(q, carry2):
            for i in range(4):
                r = q * 4 + i
                v = [in_v[r, pl.ds(k * 16, 16)] for k in range(_D // 16)]
                sq = v[0] * v[0] + v[1] * v[1] + v[2] * v[2] + v[3] * v[3]
                cs = plsc.cumsum(sq)
                tot = jnp.full((16,), cs[15], jnp.float32)
                scale = _soft_clip_scale(tot)
                for k in range(_D // 16):
                    out_v[r, pl.ds(k * 16, 16)] = (v[k] + jnp.float32(1e-15)) * scale
            return carry2

        lax.fori_loop(0, _CHUNK // 4, quad_body, 0, unroll=False)
        pltpu.sync_copy(out_v, out_hbm.at[pl.ds(base + c * _CHUNK, _CHUNK)])
        return carry

    lax.fori_loop(0, _NCHUNK, chunk_body, 0, unroll=False)


_embed_sc = functools.partial(
    pl.kernel,
    out_type=jax.ShapeDtypeStruct((_TOTAL, _D), jnp.float32),
    mesh=plsc.VectorSubcoreMesh(core_axis_name="c", subcore_axis_name="s"),
    compiler_params=pltpu.CompilerParams(needs_layout_passes=False,
                                         use_tc_tiling_on_sc=False),
    scratch_types=[
        pltpu.VMEM((_PER_W,), jnp.int32),
        pltpu.VMEM((_CHUNK, _D), jnp.float32),
        pltpu.VMEM((_CHUNK, _D), jnp.float32),
        pltpu.SemaphoreType.DMA,
    ],
)(_embed_body)


def kernel(inputs, table):
    flat_idx = inputs.reshape(_TOTAL)
    out = _embed_sc(flat_idx, table)
    return out.reshape(_B, _H, _D)


# speedup vs baseline: 1.8740x; 1.2010x over previous
import functools

import jax
import jax.numpy as jnp
import numpy as np
from jax import lax
from jax.experimental import pallas as pl
from jax.experimental.pallas import tpu as pltpu
from jax.experimental.pallas import tpu_sc as plsc

_D = 64
_B = 16384
_H = 50
_TOTAL = _B * _H
_NC, _NS = 2, 16
_NW = _NC * _NS
_PER_W = _TOTAL // _NW
_CHUNK = 128
_NCHUNK = _PER_W // _CHUNK

_K = 2.0
_IR = np.float32(2.0 * _K / (1.0 + np.sqrt(1.0 + 4.0 * _K * _K)))
_EPS = np.float32(1e-05)


def _rsqrt(x):
    i = plsc.bitcast(x, jnp.int32)
    i = jnp.int32(0x5F3759DF) - (i >> 1)
    y = plsc.bitcast(i, jnp.float32)
    for _ in range(3):
        y = y * (jnp.float32(1.5) - jnp.float32(0.5) * x * y * y)
    return y


def _soft_clip_scale(sumsq):
    y = _rsqrt(sumsq)
    norm = sumsq * y
    one = jnp.full((16,), 1.0, jnp.float32)
    return jnp.where(norm <= _IR, _IR * y,
                     jnp.where(norm >= jnp.float32(1.0), (jnp.float32(1.0) - _EPS) * y,
                               one))


def _clip_rows(in_v, out_v):
    def quad_body(q, carry2):
        for i in range(4):
            r = q * 4 + i
            v = [in_v[r, pl.ds(k * 16, 16)] for k in range(_D // 16)]
            sq = v[0] * v[0] + v[1] * v[1] + v[2] * v[2] + v[3] * v[3]
            cs = plsc.cumsum(sq)
            tot = jnp.full((16,), cs[15], jnp.float32)
            scale = _soft_clip_scale(tot)
            for k in range(_D // 16):
                out_v[r, pl.ds(k * 16, 16)] = (v[k] + jnp.float32(1e-15)) * scale
        return carry2

    lax.fori_loop(0, _CHUNK // 4, quad_body, 0, unroll=False)


def _embed_body(idx_hbm, table_hbm, out_hbm, idx_v,
                in_v0, in_v1, out_v0, out_v1,
                sem_in0, sem_in1, sem_out0, sem_out1):
    wid = lax.axis_index("s") * _NC + lax.axis_index("c")
    base = wid * _PER_W
    pltpu.sync_copy(idx_hbm.at[pl.ds(base, _PER_W)], idx_v)
    in_bufs = (in_v0, in_v1)
    out_bufs = (out_v0, out_v1)
    sem_ins = (sem_in0, sem_in1)
    sem_outs = (sem_out0, sem_out1)

    def gather_desc(c, b):
        return pltpu.make_async_copy(
            table_hbm.at[idx_v.at[pl.ds(c * _CHUNK, _CHUNK)]],
            in_bufs[b], sem_ins[b])

    def put_desc(c, b):
        return pltpu.make_async_copy(
            out_bufs[b], out_hbm.at[pl.ds(base + c * _CHUNK, _CHUNK)],
            sem_outs[b])

    gather_desc(0, 0).start()
    gather_desc(1, 1).start()

    def pair_body(p, carry):
        c0 = p * 2
        for b in range(2):
            c = c0 + b
            gather_desc(c, b).wait()

            @pl.when(c0 > 0)
            def _wait_out():
                put_desc(c, b).wait()

            _clip_rows(in_bufs[b], out_bufs[b])
            put_desc(c, b).start()

            @pl.when(c < _NCHUNK - 2)
            def _next_gather():
                gather_desc(c + 2, b).start()

        return carry

    lax.fori_loop(0, _NCHUNK // 2, pair_body, 0, unroll=False)
    for b in range(2):
        put_desc(_NCHUNK - 2 + b, b).wait()


_embed_sc = functools.partial(
    pl.kernel,
    out_type=jax.ShapeDtypeStruct((_TOTAL, _D), jnp.float32),
    mesh=plsc.VectorSubcoreMesh(core_axis_name="c", subcore_axis_name="s"),
    compiler_params=pltpu.CompilerParams(needs_layout_passes=False,
                                         use_tc_tiling_on_sc=False),
    scratch_types=[
        pltpu.VMEM((_PER_W,), jnp.int32),
        pltpu.VMEM((_CHUNK, _D), jnp.float32),
        pltpu.VMEM((_CHUNK, _D), jnp.float32),
        pltpu.VMEM((_CHUNK, _D), jnp.float32),
        pltpu.VMEM((_CHUNK, _D), jnp.float32),
        pltpu.SemaphoreType.DMA,
        pltpu.SemaphoreType.DMA,
        pltpu.SemaphoreType.DMA,
        pltpu.SemaphoreType.DMA,
    ],
)(_embed_body)


def kernel(inputs, table):
    flat_idx = inputs.reshape(_TOTAL)
    out = _embed_sc(flat_idx, table)
    return out.reshape(_B, _H, _D)
